# Initial kernel scaffold; baseline (speedup 1.0000x reference)
import jax
import jax.numpy as jnp
from jax.experimental import pallas as pl


def kernel(x, mask, emb_table, pos_table, W1, b1, W2, b2):
    raise NotImplementedError("write your pallas kernel here")


# speedup vs baseline: 2.1666x; 2.1666x over previous
import functools

import jax
import jax.numpy as jnp
from jax.experimental import pallas as pl
from jax.experimental.pallas import tpu as pltpu
from jax.experimental.pallas import tpu_sc as plsc

V_SIZE = 100000
N_POS = 50
EMB = 128
HID = 256

_GATHER_WINDOW = 128


def _sc_gather(table, idx_flat):
    num_indices = idx_flat.shape[0]
    idx2d = idx_flat.reshape(1, num_indices)
    mesh = plsc.VectorSubcoreMesh(core_axis_name="c", subcore_axis_name="s")

    @functools.partial(
        pl.kernel,
        out_type=jax.ShapeDtypeStruct((num_indices, EMB), table.dtype),
        mesh=mesh,
    )
    def gather_kernel(table_hbm, idx_hbm, out_hbm):
        def body(idx_vmem, out_vmem):
            pltpu.sync_copy(table_hbm.at[idx_vmem.at[0]], out_vmem)

        pltpu.emit_pipeline(
            body,
            grid=(num_indices // _GATHER_WINDOW,),
            in_specs=[
                pl.BlockSpec((1, _GATHER_WINDOW), index_map=lambda i: (0, i))
            ],
            out_specs=[
                pl.BlockSpec((_GATHER_WINDOW, EMB), index_map=lambda i: (i, 0))
            ],
            core_axis_name=("c", "s"),
            dimension_semantics=(pltpu.PARALLEL,),
        )(idx_hbm, out_hbm)

    return gather_kernel(table, idx2d)


def _mlp_body(x_ref, p_ref, w1_ref, b1_ref, w2_ref, b2_ref, o_ref):
    h = x_ref[...] + p_ref[...]
    a = jnp.dot(h, w1_ref[...], preferred_element_type=jnp.float32)
    a = jnp.maximum(a + b1_ref[...], 0.0)
    o = jnp.dot(a, w2_ref[...], preferred_element_type=jnp.float32)
    o_ref[...] = o + b2_ref[...]


def _mlp(gathered, pos_tiled, W1, b1, W2, b2, block_rows):
    n_rows = gathered.shape[0]
    return pl.pallas_call(
        _mlp_body,
        grid=(n_rows // block_rows,),
        in_specs=[
            pl.BlockSpec((block_rows, EMB), lambda i: (i, 0)),
            pl.BlockSpec((block_rows, EMB), lambda i: (0, 0)),
            pl.BlockSpec((EMB, HID), lambda i: (0, 0)),
            pl.BlockSpec((1, HID), lambda i: (0, 0)),
            pl.BlockSpec((HID, EMB), lambda i: (0, 0)),
            pl.BlockSpec((1, EMB), lambda i: (0, 0)),
        ],
        out_specs=pl.BlockSpec((block_rows, EMB), lambda i: (i, 0)),
        out_shape=jax.ShapeDtypeStruct((n_rows, EMB), jnp.float32),
        compiler_params=pltpu.CompilerParams(
            dimension_semantics=("parallel",)
        ),
    )(gathered, pos_tiled, W1, b1.reshape(1, HID), W2, b2.reshape(1, EMB))


def kernel(x, mask, emb_table, pos_table, W1, b1, W2, b2):
    B, N = x.shape
    xi = jnp.where(mask, V_SIZE - 1, x).astype(jnp.int32)
    gathered = _sc_gather(emb_table, xi.reshape(-1))
    block_rows = 3200
    pos_tiled = jnp.tile(pos_table, (block_rows // N, 1))
    out = _mlp(gathered, pos_tiled, W1, b1, W2, b2, block_rows)
    return out.reshape(B, N, EMB)


# speedup vs baseline: 2.1783x; 1.0054x over previous
import functools

import jax
import jax.numpy as jnp
from jax.experimental import pallas as pl
from jax.experimental.pallas import tpu as pltpu
from jax.experimental.pallas import tpu_sc as plsc

V_SIZE = 100000
N_POS = 50
EMB = 128
HID = 256

_GATHER_WINDOW = 128


def _sc_gather(table, idx_rows):
    num_windows = idx_rows.shape[0]
    num_indices = num_windows * _GATHER_WINDOW
    mesh = plsc.VectorSubcoreMesh(core_axis_name="c", subcore_axis_name="s")

    @functools.partial(
        pl.kernel,
        out_type=jax.ShapeDtypeStruct((num_indices, EMB), table.dtype),
        mesh=mesh,
    )
    def gather_kernel(table_hbm, idx_hbm, out_hbm):
        def body(idx_vmem, out_vmem):
            pltpu.sync_copy(table_hbm.at[idx_vmem.at[0]], out_vmem)

        pltpu.emit_pipeline(
            body,
            grid=(num_windows,),
            in_specs=[
                pl.BlockSpec((1, _GATHER_WINDOW), index_map=lambda i: (i, 0))
            ],
            out_specs=[
                pl.BlockSpec((_GATHER_WINDOW, EMB), index_map=lambda i: (i, 0))
            ],
            core_axis_name=("c", "s"),
            dimension_semantics=(pltpu.PARALLEL,),
        )(idx_hbm, out_hbm)

    return gather_kernel(table, idx_rows)


def _mlp_body(x_ref, p_ref, w1_ref, b1_ref, w2_ref, b2_ref, o_ref):
    h = x_ref[...] + p_ref[...]
    a = jnp.dot(h, w1_ref[...], preferred_element_type=jnp.float32)
    a = jnp.maximum(a + b1_ref[...], 0.0)
    o = jnp.dot(a, w2_ref[...], preferred_element_type=jnp.float32)
    o_ref[...] = o + b2_ref[...]


def _mlp(gathered, pos_tiled, W1, b1, W2, b2, block_rows):
    n_rows = gathered.shape[0]
    return pl.pallas_call(
        _mlp_body,
        grid=(n_rows // block_rows,),
        in_specs=[
            pl.BlockSpec((block_rows, EMB), lambda i: (i, 0)),
            pl.BlockSpec((block_rows, EMB), lambda i: (0, 0)),
            pl.BlockSpec((EMB, HID), lambda i: (0, 0)),
            pl.BlockSpec((1, HID), lambda i: (0, 0)),
            pl.BlockSpec((HID, EMB), lambda i: (0, 0)),
            pl.BlockSpec((1, EMB), lambda i: (0, 0)),
        ],
        out_specs=pl.BlockSpec((block_rows, EMB), lambda i: (i, 0)),
        out_shape=jax.ShapeDtypeStruct((n_rows, EMB), jnp.float32),
        compiler_params=pltpu.CompilerParams(
            dimension_semantics=("parallel",)
        ),
    )(gathered, pos_tiled, W1, b1.reshape(1, HID), W2, b2.reshape(1, EMB))


def kernel(x, mask, emb_table, pos_table, W1, b1, W2, b2):
    B, N = x.shape
    xi = jnp.where(mask, V_SIZE - 1, x).astype(jnp.int32)
    gathered = _sc_gather(emb_table, xi.reshape(-1, _GATHER_WINDOW))
    block_rows = 3200
    pos_tiled = jnp.tile(pos_table, (block_rows // N, 1))
    out = _mlp(gathered, pos_tiled, W1, b1, W2, b2, block_rows)
    return out.reshape(B, N, EMB)


# speedup vs baseline: 3.0928x; 1.4198x over previous
import functools

import jax
import jax.numpy as jnp
from jax.experimental import pallas as pl
from jax.experimental.pallas import tpu as pltpu
from jax.experimental.pallas import tpu_sc as plsc

V_SIZE = 100000
N_POS = 50
EMB = 128
HID = 256

_GATHER_WINDOW = 128


def _sc_gather(table, idx_rows):
    num_windows = idx_rows.shape[0]
    num_indices = num_windows * _GATHER_WINDOW
    mesh = plsc.VectorSubcoreMesh(core_axis_name="c", subcore_axis_name="s")

    @functools.partial(
        pl.kernel,
        out_type=jax.ShapeDtypeStruct((num_indices, EMB), table.dtype),
        mesh=mesh,
    )
    def gather_kernel(table_hbm, idx_hbm, out_hbm):
        def body(idx_vmem, out_vmem):
            pltpu.sync_copy(table_hbm.at[idx_vmem.at[0]], out_vmem)

        pltpu.emit_pipeline(
            body,
            grid=(num_windows,),
            in_specs=[
                pl.BlockSpec((1, _GATHER_WINDOW), index_map=lambda i: (i, 0))
            ],
            out_specs=[
                pl.BlockSpec((_GATHER_WINDOW, EMB), index_map=lambda i: (i, 0))
            ],
            core_axis_name=("c", "s"),
            dimension_semantics=(pltpu.PARALLEL,),
        )(idx_hbm, out_hbm)

    return gather_kernel(table, idx_rows)


def _mlp_body(x_ref, p_ref, w1_ref, b1_ref, w2_ref, b2_ref, o_ref):
    h = x_ref[...] + p_ref[...]
    a = jnp.dot(h, w1_ref[...], preferred_element_type=jnp.float32)
    a = jnp.maximum(a + b1_ref[...], 0.0)
    o = jnp.dot(a, w2_ref[...], preferred_element_type=jnp.float32)
    o = o + b2_ref[...]
    o_ref[...] = o.reshape(o_ref.shape)


def _mlp(gathered, pos_tiled, W1, b1, W2, b2, block_rows):
    n_rows = gathered.shape[0]
    batch = n_rows // N_POS
    bb = block_rows // N_POS
    return pl.pallas_call(
        _mlp_body,
        grid=(n_rows // block_rows,),
        in_specs=[
            pl.BlockSpec((block_rows, EMB), lambda i: (i, 0)),
            pl.BlockSpec((block_rows, EMB), lambda i: (0, 0)),
            pl.BlockSpec((EMB, HID), lambda i: (0, 0)),
            pl.BlockSpec((1, HID), lambda i: (0, 0)),
            pl.BlockSpec((HID, EMB), lambda i: (0, 0)),
            pl.BlockSpec((1, EMB), lambda i: (0, 0)),
        ],
        out_specs=pl.BlockSpec((bb, N_POS, EMB), lambda i: (i, 0, 0)),
        out_shape=jax.ShapeDtypeStruct((batch, N_POS, EMB), jnp.float32),
        compiler_params=pltpu.CompilerParams(
            dimension_semantics=("parallel",)
        ),
    )(gathered, pos_tiled, W1, b1.reshape(1, HID), W2, b2.reshape(1, EMB))


def kernel(x, mask, emb_table, pos_table, W1, b1, W2, b2):
    B, N = x.shape
    xi = jnp.where(mask, V_SIZE - 1, x).astype(jnp.int32)
    gathered = _sc_gather(emb_table, xi.reshape(-1, _GATHER_WINDOW))
    block_rows = 3200
    pos_tiled = jnp.tile(pos_table, (block_rows // N, 1))
    return _mlp(gathered, pos_tiled, W1, b1, W2, b2, block_rows)
